# baseline (speedup 1.0000x reference)
import jax
import jax.numpy as jnp
import numpy as np
from jax.experimental import pallas as pl
from jax.experimental.pallas import tpu as pltpu

_TINY = np.float32(np.finfo(np.float32).tiny)
_ROT_A = (13, 15, 26, 6)
_ROT_B = (17, 29, 16, 24)


def _threefry_bits(n):
    k0 = 0
    k1 = 42
    k2 = 0 ^ 42 ^ 0x1BD11BDA

    def rotl(x, r):
        return (x << jnp.uint32(r)) | (x >> jnp.uint32(32 - r))

    x1 = n + jnp.uint32(k1)
    x0 = x1
    x1 = rotl(x1, _ROT_A[0]) ^ x0
    for r in _ROT_A[1:]:
        x0 = x0 + x1
        x1 = rotl(x1, r)
        x1 = x1 ^ x0
    x0 = x0 + jnp.uint32(k1)
    x1 = x1 + jnp.uint32((k2 + 1) & 0xFFFFFFFF)
    sched = (
        (_ROT_B, k2, k0, 2),
        (_ROT_A, k0, k1, 3),
        (_ROT_B, k1, k2, 4),
        (_ROT_A, k2, k0, 5),
    )
    for rots, ka, kb, i in sched:
        for r in rots:
            x0 = x0 + x1
            x1 = rotl(x1, r)
            x1 = x1 ^ x0
        x0 = x0 + jnp.uint32(ka)
        x1 = x1 + jnp.uint32((kb + i) & 0xFFFFFFFF)
    return x0 ^ x1


def _score_chunk(p_c, gflat, total):
    bits = _threefry_bits(gflat.astype(jnp.uint32))
    fbits = (bits >> jnp.uint32(9)) | jnp.uint32(0x3F800000)
    floats = (jax.lax.bitcast_convert_type(fbits, jnp.float32)
              - jnp.float32(1.0))
    u = jnp.maximum(jnp.float32(_TINY), floats)
    gumbel = -jnp.log(-jnp.log(u))
    return gumbel + jnp.log(p_c / total + jnp.float32(1e-30))


def _rows_kernel(p_ref, out_ref, *, v, rpb, chunk, unroll):
    pid = pl.program_id(0)
    p = p_ref[...]
    total = jnp.sum(p, axis=1, keepdims=True)
    row_i = jax.lax.broadcasted_iota(jnp.int32, (rpb, 1), 0)
    nbase = (pid * rpb + row_i) * v
    imax = jnp.int32(2**31 - 1)

    lane_i = jax.lax.broadcasted_iota(jnp.int32, (rpb, chunk), 1)
    base0 = nbase + lane_i

    n_full = v // chunk
    n_outer = n_full // unroll

    def one_chunk(off, m, idx):
        p_c = p_ref[:, pl.ds(off, chunk)]
        gflat = base0 + off
        score = _score_chunk(p_c, gflat, total)
        better = score > m
        m = jnp.where(better, score, m)
        idx = jnp.where(better, gflat, idx)
        return m, idx

    def body(k, carry):
        m, idx = carry
        for j in range(unroll):
            m, idx = one_chunk(k * (chunk * unroll) + j * chunk, m, idx)
        return m, idx

    m0 = jnp.full((rpb, chunk), -jnp.inf, jnp.float32)
    i0 = jnp.full((rpb, chunk), imax, jnp.int32)
    m, idx = jax.lax.fori_loop(0, n_outer, body, (m0, i0))
    for j in range(n_outer * unroll, n_full):
        m, idx = one_chunk(j * chunk, m, idx)

    best = jnp.max(m, axis=1, keepdims=True)
    ans = jnp.min(jnp.where(m == best, idx, imax),
                  axis=1, keepdims=True)

    tail_w = v - n_full * chunk
    if tail_w:
        tlane = jax.lax.broadcasted_iota(jnp.int32, (rpb, tail_w), 1)
        tflat = nbase + n_full * chunk + tlane
        tscore = _score_chunk(p_ref[:, n_full * chunk:v], tflat, total)
        tbest = jnp.max(tscore, axis=1, keepdims=True)
        tans = jnp.min(jnp.where(tscore == tbest, tflat, imax),
                       axis=1, keepdims=True)
        ans = jnp.where(
            tbest > best, tans,
            jnp.where(tbest == best, jnp.minimum(ans, tans), ans))

    out_ref[...] = ans - nbase


def kernel(probabilities):
    b, s, v = probabilities.shape
    rows = b * s
    p2 = probabilities.reshape(rows, v)

    chunk = 512
    rpb = 32
    unroll = 12
    out = pl.pallas_call(
        lambda p_ref, out_ref: _rows_kernel(p_ref, out_ref, v=v, rpb=rpb,
                                            chunk=chunk, unroll=unroll),
        grid=(rows // rpb,),
        in_specs=[pl.BlockSpec((rpb, v), lambda i: (i, 0))],
        out_specs=pl.BlockSpec((rpb, 1), lambda i: (i, 0)),
        out_shape=jax.ShapeDtypeStruct((rows, 1), jnp.int32),
        compiler_params=pltpu.CompilerParams(
            dimension_semantics=("parallel",)),
    )(p2)
    return out.reshape(b, s)


# speedup vs baseline: 1.0323x; 1.0323x over previous
import jax
import jax.numpy as jnp
import numpy as np
from jax.experimental import pallas as pl
from jax.experimental.pallas import tpu as pltpu

_TINY = np.float32(np.finfo(np.float32).tiny)
_ROT_A = (13, 15, 26, 6)
_ROT_B = (17, 29, 16, 24)


def _threefry_bits(n):
    k0 = 0
    k1 = 42
    k2 = 0 ^ 42 ^ 0x1BD11BDA

    def rotl(x, r):
        return (x << jnp.uint32(r)) | (x >> jnp.uint32(32 - r))

    x1 = n + jnp.uint32(k1)
    x0 = x1
    x1 = rotl(x1, _ROT_A[0]) ^ x0
    for r in _ROT_A[1:]:
        x0 = x0 + x1
        x1 = rotl(x1, r)
        x1 = x1 ^ x0
    x0 = x0 + jnp.uint32(k1)
    x1 = x1 + jnp.uint32((k2 + 1) & 0xFFFFFFFF)
    sched = (
        (_ROT_B, k2, k0, 2),
        (_ROT_A, k0, k1, 3),
        (_ROT_B, k1, k2, 4),
        (_ROT_A, k2, k0, 5),
    )
    for rots, ka, kb, i in sched:
        for r in rots:
            x0 = x0 + x1
            x1 = rotl(x1, r)
            x1 = x1 ^ x0
        x0 = x0 + jnp.uint32(ka)
        x1 = x1 + jnp.uint32((kb + i) & 0xFFFFFFFF)
    return x0 ^ x1


def _score_chunk(p_c, gflat, total):
    bits = _threefry_bits(gflat.astype(jnp.uint32))
    fbits = (bits >> jnp.uint32(9)) | jnp.uint32(0x3F800000)
    u = (jax.lax.bitcast_convert_type(fbits, jnp.float32)
         - jnp.float32(1.0))
    gumbel = -jnp.log(-jnp.log(u))
    return gumbel + jnp.log(p_c / total)


def _rows_kernel(p_ref, out_ref, *, v, rpb, chunk, unroll):
    pid = pl.program_id(0)
    p = p_ref[...]
    total = jnp.sum(p, axis=1, keepdims=True)
    row_i = jax.lax.broadcasted_iota(jnp.int32, (rpb, 1), 0)
    nbase = (pid * rpb + row_i) * v
    imax = jnp.int32(2**31 - 1)

    lane_i = jax.lax.broadcasted_iota(jnp.int32, (rpb, chunk), 1)
    base0 = nbase + lane_i

    n_full = v // chunk
    n_outer = n_full // unroll

    def one_chunk(off, m, idx):
        p_c = p_ref[:, pl.ds(off, chunk)]
        gflat = base0 + off
        score = _score_chunk(p_c, gflat, total)
        better = score > m
        m = jnp.where(better, score, m)
        idx = jnp.where(better, gflat, idx)
        return m, idx

    def body(k, carry):
        m, idx = carry
        for j in range(unroll):
            m, idx = one_chunk(k * (chunk * unroll) + j * chunk, m, idx)
        return m, idx

    m0 = jnp.full((rpb, chunk), -jnp.inf, jnp.float32)
    i0 = jnp.full((rpb, chunk), imax, jnp.int32)
    m, idx = jax.lax.fori_loop(0, n_outer, body, (m0, i0))
    for j in range(n_outer * unroll, n_full):
        m, idx = one_chunk(j * chunk, m, idx)

    best = jnp.max(m, axis=1, keepdims=True)
    ans = jnp.min(jnp.where(m == best, idx, imax),
                  axis=1, keepdims=True)

    tail_w = v - n_full * chunk
    if tail_w:
        tlane = jax.lax.broadcasted_iota(jnp.int32, (rpb, tail_w), 1)
        tflat = nbase + n_full * chunk + tlane
        tscore = _score_chunk(p_ref[:, n_full * chunk:v], tflat, total)
        tbest = jnp.max(tscore, axis=1, keepdims=True)
        tans = jnp.min(jnp.where(tscore == tbest, tflat, imax),
                       axis=1, keepdims=True)
        ans = jnp.where(
            tbest > best, tans,
            jnp.where(tbest == best, jnp.minimum(ans, tans), ans))

    out_ref[...] = ans - nbase


def kernel(probabilities):
    b, s, v = probabilities.shape
    rows = b * s
    p2 = probabilities.reshape(rows, v)

    chunk = 512
    rpb = 16
    unroll = 12
    out = pl.pallas_call(
        lambda p_ref, out_ref: _rows_kernel(p_ref, out_ref, v=v, rpb=rpb,
                                            chunk=chunk, unroll=unroll),
        grid=(rows // rpb,),
        in_specs=[pl.BlockSpec((rpb, v), lambda i: (i, 0))],
        out_specs=pl.BlockSpec((rpb, 1), lambda i: (i, 0)),
        out_shape=jax.ShapeDtypeStruct((rows, 1), jnp.int32),
        compiler_params=pltpu.CompilerParams(
            dimension_semantics=("parallel",)),
    )(p2)
    return out.reshape(b, s)


# speedup vs baseline: 1.0377x; 1.0052x over previous
import jax
import jax.numpy as jnp
import numpy as np
from jax.experimental import pallas as pl
from jax.experimental.pallas import tpu as pltpu

_TINY = np.float32(np.finfo(np.float32).tiny)
_ROT_A = (13, 15, 26, 6)
_ROT_B = (17, 29, 16, 24)


def _threefry_bits(x1):
    k0 = 0
    k1 = 42
    k2 = 0 ^ 42 ^ 0x1BD11BDA

    def rotl(x, r):
        return (x << jnp.uint32(r)) | (x >> jnp.uint32(32 - r))

    x0 = x1
    x1 = rotl(x1, _ROT_A[0]) ^ x0
    for r in _ROT_A[1:]:
        x0 = x0 + x1
        x1 = rotl(x1, r)
        x1 = x1 ^ x0
    x0 = x0 + jnp.uint32(k1)
    x1 = x1 + jnp.uint32((k2 + 1) & 0xFFFFFFFF)
    sched = (
        (_ROT_B, k2, k0, 2),
        (_ROT_A, k0, k1, 3),
        (_ROT_B, k1, k2, 4),
        (_ROT_A, k2, k0, 5),
    )
    for rots, ka, kb, i in sched:
        for r in rots:
            x0 = x0 + x1
            x1 = rotl(x1, r)
            x1 = x1 ^ x0
        x0 = x0 + jnp.uint32(ka)
        x1 = x1 + jnp.uint32((kb + i) & 0xFFFFFFFF)
    return x0 ^ x1


def _score_chunk(p_c, gflat, total):
    bits = _threefry_bits(gflat.astype(jnp.uint32))
    fbits = (bits >> jnp.uint32(9)) | jnp.uint32(0x3F800000)
    u = (jax.lax.bitcast_convert_type(fbits, jnp.float32)
         - jnp.float32(1.0))
    gumbel = -jnp.log(-jnp.log(u))
    return gumbel + jnp.log(p_c / total)


def _rows_kernel(p_ref, out_ref, *, v, rpb, chunk, unroll):
    pid = pl.program_id(0)
    p = p_ref[...]
    total = jnp.sum(p, axis=1, keepdims=True)
    row_i = jax.lax.broadcasted_iota(jnp.int32, (rpb, 1), 0)
    nbase = (pid * rpb + row_i) * v
    imax = jnp.int32(2**31 - 1)

    lane_i = jax.lax.broadcasted_iota(jnp.int32, (rpb, chunk), 1)
    base0 = nbase + lane_i + jnp.int32(42)

    n_full = v // chunk
    n_outer = n_full // unroll

    def one_chunk(off, m, idx):
        p_c = p_ref[:, pl.ds(off, chunk)]
        gflat = base0 + off
        score = _score_chunk(p_c, gflat, total)
        better = score > m
        m = jnp.where(better, score, m)
        idx = jnp.where(better, gflat, idx)
        return m, idx

    def body(k, carry):
        m, idx = carry
        for j in range(unroll):
            m, idx = one_chunk(k * (chunk * unroll) + j * chunk, m, idx)
        return m, idx

    m0 = jnp.full((rpb, chunk), -jnp.inf, jnp.float32)
    i0 = jnp.full((rpb, chunk), imax, jnp.int32)
    m, idx = jax.lax.fori_loop(0, n_outer, body, (m0, i0))
    for j in range(n_outer * unroll, n_full):
        m, idx = one_chunk(j * chunk, m, idx)

    best = jnp.max(m, axis=1, keepdims=True)
    ans = jnp.min(jnp.where(m == best, idx, imax),
                  axis=1, keepdims=True)

    tail_w = v - n_full * chunk
    if tail_w:
        tlane = jax.lax.broadcasted_iota(jnp.int32, (rpb, tail_w), 1)
        tflat = nbase + n_full * chunk + tlane + jnp.int32(42)
        tscore = _score_chunk(p_ref[:, n_full * chunk:v], tflat, total)
        tbest = jnp.max(tscore, axis=1, keepdims=True)
        tans = jnp.min(jnp.where(tscore == tbest, tflat, imax),
                       axis=1, keepdims=True)
        ans = jnp.where(
            tbest > best, tans,
            jnp.where(tbest == best, jnp.minimum(ans, tans), ans))

    out_ref[...] = ans - nbase - jnp.int32(42)


def kernel(probabilities):
    b, s, v = probabilities.shape
    rows = b * s
    p2 = probabilities.reshape(rows, v)

    chunk = 512
    rpb = 16
    unroll = 13
    out = pl.pallas_call(
        lambda p_ref, out_ref: _rows_kernel(p_ref, out_ref, v=v, rpb=rpb,
                                            chunk=chunk, unroll=unroll),
        grid=(rows // rpb,),
        in_specs=[pl.BlockSpec((rpb, v), lambda i: (i, 0))],
        out_specs=pl.BlockSpec((rpb, 1), lambda i: (i, 0)),
        out_shape=jax.ShapeDtypeStruct((rows, 1), jnp.int32),
        compiler_params=pltpu.CompilerParams(
            dimension_semantics=("parallel",)),
    )(p2)
    return out.reshape(b, s)


# speedup vs baseline: 1.0388x; 1.0011x over previous
import jax
import jax.numpy as jnp
import numpy as np
from jax.experimental import pallas as pl
from jax.experimental.pallas import tpu as pltpu

_TINY = np.float32(np.finfo(np.float32).tiny)
_ROT_A = (13, 15, 26, 6)
_ROT_B = (17, 29, 16, 24)


def _threefry_bits(x1):
    k0 = 0
    k1 = 42
    k2 = 0 ^ 42 ^ 0x1BD11BDA

    def rotl(x, r):
        return (x << jnp.uint32(r)) | (x >> jnp.uint32(32 - r))

    x0 = x1
    x1 = rotl(x1, _ROT_A[0]) ^ x0
    for r in _ROT_A[1:]:
        x0 = x0 + x1
        x1 = rotl(x1, r)
        x1 = x1 ^ x0
    x0 = x0 + jnp.uint32(k1)
    x1 = x1 + jnp.uint32((k2 + 1) & 0xFFFFFFFF)
    sched = (
        (_ROT_B, k2, k0, 2),
        (_ROT_A, k0, k1, 3),
        (_ROT_B, k1, k2, 4),
        (_ROT_A, k2, k0, 5),
    )
    for rots, ka, kb, i in sched:
        for r in rots:
            x0 = x0 + x1
            x1 = rotl(x1, r)
            x1 = x1 ^ x0
        x0 = x0 + jnp.uint32(ka)
        x1 = x1 + jnp.uint32((kb + i) & 0xFFFFFFFF)
    return x0 ^ x1


def _score_chunk(p_c, gflat, total):
    bits = _threefry_bits(gflat.astype(jnp.uint32))
    fbits = (bits >> jnp.uint32(9)) | jnp.uint32(0x3F800000)
    u = (jax.lax.bitcast_convert_type(fbits, jnp.float32)
         - jnp.float32(1.0))
    gumbel = -jnp.log(-jnp.log(u))
    return gumbel + jnp.log(p_c / total)


def _rows_kernel(p_ref, out_ref, *, v, rpb, chunk, unroll):
    pid = pl.program_id(0)
    p = p_ref[...]
    total = jnp.sum(p, axis=1, keepdims=True)
    row_i = jax.lax.broadcasted_iota(jnp.int32, (rpb, 1), 0)
    nbase = (pid * rpb + row_i) * v
    imax = jnp.int32(2**31 - 1)

    lane_i = jax.lax.broadcasted_iota(jnp.int32, (rpb, chunk), 1)
    base0 = nbase + lane_i + jnp.int32(42)

    n_full = v // chunk
    n_outer = n_full // unroll

    def one_chunk(off, m, idx):
        p_c = p_ref[:, pl.ds(off, chunk)]
        gflat = base0 + off
        score = _score_chunk(p_c, gflat, total)
        better = score > m
        m = jnp.where(better, score, m)
        idx = jnp.where(better, gflat, idx)
        return m, idx

    def body(k, carry):
        m, idx = carry
        for j in range(unroll):
            m, idx = one_chunk(k * (chunk * unroll) + j * chunk, m, idx)
        return m, idx

    m0 = jnp.full((rpb, chunk), -jnp.inf, jnp.float32)
    i0 = jnp.full((rpb, chunk), imax, jnp.int32)
    m, idx = jax.lax.fori_loop(0, n_outer, body, (m0, i0))
    for j in range(n_outer * unroll, n_full):
        m, idx = one_chunk(j * chunk, m, idx)

    best = jnp.max(m, axis=1, keepdims=True)
    ans = jnp.min(jnp.where(m == best, idx, imax),
                  axis=1, keepdims=True)

    tail_w = v - n_full * chunk
    if tail_w:
        tlane = jax.lax.broadcasted_iota(jnp.int32, (rpb, tail_w), 1)
        tflat = nbase + n_full * chunk + tlane + jnp.int32(42)
        tscore = _score_chunk(p_ref[:, n_full * chunk:v], tflat, total)
        tbest = jnp.max(tscore, axis=1, keepdims=True)
        tans = jnp.min(jnp.where(tscore == tbest, tflat, imax),
                       axis=1, keepdims=True)
        ans = jnp.where(
            tbest > best, tans,
            jnp.where(tbest == best, jnp.minimum(ans, tans), ans))

    out_ref[...] = ans - nbase - jnp.int32(42)


def kernel(probabilities):
    b, s, v = probabilities.shape
    rows = b * s
    p2 = probabilities.reshape(rows, v)

    chunk = 512
    rpb = 16
    unroll = 15
    out = pl.pallas_call(
        lambda p_ref, out_ref: _rows_kernel(p_ref, out_ref, v=v, rpb=rpb,
                                            chunk=chunk, unroll=unroll),
        grid=(rows // rpb,),
        in_specs=[pl.BlockSpec((rpb, v), lambda i: (i, 0))],
        out_specs=pl.BlockSpec((rpb, 1), lambda i: (i, 0)),
        out_shape=jax.ShapeDtypeStruct((rows, 1), jnp.int32),
        compiler_params=pltpu.CompilerParams(
            dimension_semantics=("parallel",)),
    )(p2)
    return out.reshape(b, s)
